# baseline (speedup 1.0000x reference)
import functools

import jax
import jax.numpy as jnp
import numpy as np
from jax import lax
from jax.experimental import pallas as pl
from jax.experimental.pallas import tpu as pltpu
from jax.experimental.pallas import tpu_sc as plsc

N = 10000
E = 320000
D = 128
DP = D // 2
NC = 2
NS = 16
NW = NC * NS
EPT = E // NW
CH = 80
NCHUNK = EPT // CH
DEPTH = 2
NPROC = 126
NPADC = NPROC + DEPTH
RPT = 624
RTAIL = N - NS * RPT

_PERM = np.stack([np.arange(0, 64), np.arange(64, 128)], 1).ravel()

_mesh = plsc.VectorSubcoreMesh(core_axis_name="c", subcore_axis_name="s")


@functools.partial(
    pl.kernel,
    mesh=_mesh,
    compiler_params=pltpu.CompilerParams(
        use_tc_tiling_on_sc=False, needs_layout_passes=False),
    out_type=jax.ShapeDtypeStruct((NC, N, D), jnp.bfloat16),
    scratch_types=(
        [pltpu.VMEM((NPADC, CH), jnp.int32)]
        + [pltpu.VMEM((NPROC, CH), jnp.int32)]
        + [pltpu.VMEM((CH, D), jnp.float32) for _ in range(DEPTH)]
        + [pltpu.VMEM((CH, D), jnp.bfloat16) for _ in range(DEPTH)]
        + [pltpu.VMEM((NPROC, CH), jnp.float32)]
        + [pltpu.VMEM_SHARED((N, D), jnp.bfloat16)]
        + [pltpu.SemaphoreType.DMA] * (2 * DEPTH)
    ),
)
def _aggregate(xs_hbm, src_hbm, dst_hbm, ew_hbm, out_hbm, *refs):
    src_v = refs[0]
    dst_v = refs[1]
    gb = refs[2:2 + DEPTH]
    sb = refs[2 + DEPTH:2 + 2 * DEPTH]
    ew_v = refs[2 + 2 * DEPTH]
    acc = refs[3 + 2 * DEPTH]
    gsem = refs[4 + 2 * DEPTH:4 + 3 * DEPTH]
    ssem = refs[4 + 3 * DEPTH:4 + 4 * DEPTH]

    cc = lax.axis_index("c")
    ss = lax.axis_index("s")
    t = cc * NS + ss

    pltpu.sync_copy(src_hbm.at[t], src_v)
    pltpu.sync_copy(dst_hbm.at[t], dst_v)
    pltpu.sync_copy(ew_hbm.at[t], ew_v)

    zb = jnp.zeros((32,), jnp.bfloat16)

    def _zrow(i, _):
        for b in range(DEPTH):
            for j in range(D // 32):
                sb[b][i, pl.ds(j * 32, 32)] = zb
        return 0
    lax.fori_loop(0, CH, _zrow, 0)

    for k in range(7):
        pltpu.sync_copy(sb[0], acc.at[pl.ds(ss * RPT + k * CH, CH)])
    pltpu.sync_copy(sb[0].at[pl.ds(0, RPT - 7 * CH)],
                    acc.at[pl.ds(ss * RPT + 7 * CH, RPT - 7 * CH)])

    @pl.when(ss == NS - 1)
    def _zero_tail():
        pltpu.sync_copy(sb[0].at[pl.ds(0, RTAIL)],
                        acc.at[pl.ds(NS * RPT, RTAIL)])

    plsc.subcore_barrier()

    def _scale(gbuf, sbuf, ci):
        def _grp(g, _):
            wvec = ew_v[ci, pl.ds(g * 16, 16)]
            for l in range(16):
                e = g * 16 + l
                w = wvec[l]
                for h in range(4):
                    vlo = gbuf[e, pl.ds(h * 16, 16)]
                    vhi = gbuf[e, pl.ds(64 + h * 16, 16)]
                    sbuf[e, pl.ds(h * 32, 32)] = plsc.pack(
                        vlo * w, vhi * w, format=plsc.PackFormat.INTERLEAVED)
            return 0
        lax.fori_loop(0, CH // 16, _grp, 0)

    zpad = NPROC - 1
    for b in range(DEPTH):
        pltpu.async_copy(xs_hbm.at[src_v.at[b]], gb[b], gsem[b])
        pltpu.async_copy(sb[b], acc.at[dst_v.at[zpad]], ssem[b], add=True)

    def _round(i, _):
        for b in range(DEPTH):
            c = i * DEPTH + b
            pltpu.make_async_copy(xs_hbm.at[src_v.at[c]], gb[b],
                                  gsem[b]).wait()
            pltpu.make_async_copy(sb[b], acc.at[dst_v.at[c]], ssem[b]).wait()
            _scale(gb[b], sb[b], c)
            pltpu.async_copy(xs_hbm.at[src_v.at[c + DEPTH]], gb[b], gsem[b])
            pltpu.async_copy(sb[b], acc.at[dst_v.at[c]], ssem[b], add=True)
        return 0

    lax.fori_loop(0, NPROC // DEPTH, _round, 0)
    for b in range(DEPTH):
        pltpu.make_async_copy(xs_hbm.at[src_v.at[0]], gb[b], gsem[b]).wait()
        pltpu.make_async_copy(sb[b], acc.at[dst_v.at[zpad]], ssem[b]).wait()
    plsc.subcore_barrier()

    pltpu.sync_copy(acc.at[pl.ds(ss * RPT, RPT)],
                    out_hbm.at[cc, pl.ds(ss * RPT, RPT)])

    @pl.when(ss == NS - 1)
    def _write_tail():
        pltpu.sync_copy(acc.at[pl.ds(NS * RPT, RTAIL)],
                        out_hbm.at[cc, pl.ds(NS * RPT, RTAIL)])


_BM = 1000


def _mm_body(p_ref, w_ref, b_ref, o_ref):
    o_ref[...] = (
        jnp.dot(p_ref[0], w_ref[...], preferred_element_type=jnp.float32)
        + jnp.dot(p_ref[1], w_ref[...], preferred_element_type=jnp.float32)
        + b_ref[...]
    )


def _finish(partials, Wp, b2):
    return pl.pallas_call(
        _mm_body,
        grid=(N // _BM,),
        in_specs=[
            pl.BlockSpec((NC, _BM, D), lambda i: (0, i, 0)),
            pl.BlockSpec((D, D), lambda i: (0, 0)),
            pl.BlockSpec((1, D), lambda i: (0, 0)),
        ],
        out_specs=pl.BlockSpec((_BM, D), lambda i: (i, 0)),
        out_shape=jax.ShapeDtypeStruct((N, D), jnp.float32),
    )(partials, Wp, b2)


def kernel(X, edge_index, edge_weight, W, b):
    src = jnp.pad(edge_index[0].astype(jnp.int32).reshape(NW, NCHUNK, CH),
                  ((0, 0), (0, NPADC - NCHUNK), (0, 0)))
    dst = jnp.pad(edge_index[1].astype(jnp.int32).reshape(NW, NCHUNK, CH),
                  ((0, 0), (0, NPROC - NCHUNK), (0, 0)))
    ew = jnp.pad(edge_weight.reshape(NW, NCHUNK, CH),
                 ((0, 0), (0, NPROC - NCHUNK), (0, 0)))
    partials = _aggregate(X, src, dst, ew)
    wp = W[_PERM]
    return _finish(partials, wp, b.reshape(1, D))


# speedup vs baseline: 1.6684x; 1.6684x over previous
import functools

import jax
import jax.numpy as jnp
import numpy as np
from jax import lax
from jax.experimental import pallas as pl
from jax.experimental.pallas import tpu as pltpu
from jax.experimental.pallas import tpu_sc as plsc

N = 10000
E = 320000
D = 128
DH = D // 2
NC = 2
NS = 16
EPT = E // NS
CH = 80
NCHUNK = EPT // CH
RPT = 624
RTAIL = N - NS * RPT

_LPERM = np.concatenate([
    np.arange(0, 32, 2), np.arange(1, 32, 2),
    np.arange(32, 64, 2), np.arange(33, 64, 2),
])

_mesh = plsc.VectorSubcoreMesh(core_axis_name="c", subcore_axis_name="s")


@functools.partial(
    pl.kernel,
    mesh=_mesh,
    compiler_params=pltpu.CompilerParams(
        use_tc_tiling_on_sc=False, needs_layout_passes=False),
    out_type=jax.ShapeDtypeStruct((NC, N, DH), jnp.float32),
    scratch_types=[
        pltpu.VMEM((NCHUNK + 2, CH), jnp.int32),
        pltpu.VMEM((NCHUNK, CH), jnp.int32),
        pltpu.VMEM((CH, DH), jnp.bfloat16),
        pltpu.VMEM((CH, DH), jnp.bfloat16),
        pltpu.VMEM((CH, DH), jnp.float32),
        pltpu.VMEM((CH, DH), jnp.float32),
        pltpu.VMEM((NCHUNK, CH), jnp.float32),
        pltpu.VMEM_SHARED((N, DH), jnp.float32),
        pltpu.SemaphoreType.DMA,
        pltpu.SemaphoreType.DMA,
        pltpu.SemaphoreType.DMA,
        pltpu.SemaphoreType.DMA,
    ],
)
def _aggregate(x2_hbm, src2_hbm, dst_hbm, ew_hbm, out_hbm,
               src_v, dst_v, g0_v, g1_v, s0_v, s1_v, ew_v,
               acc, sem0, sem1, ssem0, ssem1):
    cc = lax.axis_index("c")
    ss = lax.axis_index("s")

    pltpu.sync_copy(src2_hbm.at[cc, ss], src_v.at[pl.ds(0, NCHUNK)])
    pltpu.sync_copy(dst_hbm.at[ss], dst_v)
    pltpu.sync_copy(ew_hbm.at[ss], ew_v)

    def _zpad(i, _):
        for j in range(CH // 16):
            src_v[NCHUNK + i, pl.ds(j * 16, 16)] = jnp.zeros((16,), jnp.int32)
        return 0
    lax.fori_loop(0, 2, _zpad, 0)

    def _zrow(i, _):
        for j in range(DH // 16):
            s0_v[i, pl.ds(j * 16, 16)] = jnp.zeros((16,), jnp.float32)
            s1_v[i, pl.ds(j * 16, 16)] = jnp.zeros((16,), jnp.float32)
        return 0
    lax.fori_loop(0, CH, _zrow, 0)
    for k in range(7):
        pltpu.sync_copy(s0_v, acc.at[pl.ds(ss * RPT + k * CH, CH)])
    pltpu.sync_copy(s0_v.at[pl.ds(0, RPT - 7 * CH)],
                    acc.at[pl.ds(ss * RPT + 7 * CH, RPT - 7 * CH)])

    @pl.when(ss == NS - 1)
    def _zero_tail():
        pltpu.sync_copy(s0_v.at[pl.ds(0, RTAIL)],
                        acc.at[pl.ds(NS * RPT, RTAIL)])

    plsc.subcore_barrier()

    himask = jnp.int32(-65536)

    def _scale(gbuf, sbuf, ci):
        def _grp(g, _):
            wvec = ew_v[ci, pl.ds(g * 16, 16)]
            for l in range(16):
                e = g * 16 + l
                w = wvec[l]
                for h in range(DH // 32):
                    v = plsc.bitcast(gbuf[e, pl.ds(h * 32, 32)], jnp.int32)
                    lo = plsc.bitcast(v << 16, jnp.float32)
                    hi = plsc.bitcast(v & himask, jnp.float32)
                    sbuf[e, pl.ds(h * 32, 16)] = lo * w
                    sbuf[e, pl.ds(h * 32 + 16, 16)] = hi * w
            return 0
        lax.fori_loop(0, CH // 16, _grp, 0)

    pltpu.async_copy(x2_hbm.at[src_v.at[0]], g0_v, sem0)
    pltpu.async_copy(x2_hbm.at[src_v.at[1]], g1_v, sem1)
    pltpu.async_copy(s0_v, acc.at[dst_v.at[0]], ssem0, add=True)
    pltpu.async_copy(s1_v, acc.at[dst_v.at[1]], ssem1, add=True)

    def _pair(i, _):
        c0 = i * 2
        pltpu.make_async_copy(x2_hbm.at[src_v.at[c0]], g0_v, sem0).wait()
        pltpu.make_async_copy(s0_v, acc.at[dst_v.at[c0]], ssem0).wait()
        _scale(g0_v, s0_v, c0)
        pltpu.async_copy(x2_hbm.at[src_v.at[c0 + 2]], g0_v, sem0)
        pltpu.async_copy(s0_v, acc.at[dst_v.at[c0]], ssem0, add=True)

        pltpu.make_async_copy(x2_hbm.at[src_v.at[c0 + 1]], g1_v, sem1).wait()
        pltpu.make_async_copy(s1_v, acc.at[dst_v.at[c0 + 1]], ssem1).wait()
        _scale(g1_v, s1_v, c0 + 1)
        pltpu.async_copy(x2_hbm.at[src_v.at[c0 + 3]], g1_v, sem1)
        pltpu.async_copy(s1_v, acc.at[dst_v.at[c0 + 1]], ssem1, add=True)
        return 0

    lax.fori_loop(0, NCHUNK // 2, _pair, 0)
    pltpu.make_async_copy(s0_v, acc.at[dst_v.at[0]], ssem0).wait()
    pltpu.make_async_copy(s1_v, acc.at[dst_v.at[1]], ssem1).wait()
    pltpu.make_async_copy(x2_hbm.at[src_v.at[NCHUNK]], g0_v, sem0).wait()
    pltpu.make_async_copy(x2_hbm.at[src_v.at[NCHUNK + 1]], g1_v, sem1).wait()
    plsc.subcore_barrier()

    pltpu.sync_copy(acc.at[pl.ds(ss * RPT, RPT)],
                    out_hbm.at[cc, pl.ds(ss * RPT, RPT)])

    @pl.when(ss == NS - 1)
    def _write_tail():
        pltpu.sync_copy(acc.at[pl.ds(NS * RPT, RTAIL)],
                        out_hbm.at[cc, pl.ds(NS * RPT, RTAIL)])


_BM = 1000


def _mm_body(p_ref, w_ref, b_ref, o_ref):
    o_ref[...] = (
        jnp.dot(p_ref[0], w_ref[0], preferred_element_type=jnp.float32)
        + jnp.dot(p_ref[1], w_ref[1], preferred_element_type=jnp.float32)
        + b_ref[...]
    )


def _finish(partials, W2, b2):
    return pl.pallas_call(
        _mm_body,
        grid=(N // _BM,),
        in_specs=[
            pl.BlockSpec((NC, _BM, DH), lambda i: (0, i, 0)),
            pl.BlockSpec((NC, DH, D), lambda i: (0, 0, 0)),
            pl.BlockSpec((1, D), lambda i: (0, 0)),
        ],
        out_specs=pl.BlockSpec((_BM, D), lambda i: (i, 0)),
        out_shape=jax.ShapeDtypeStruct((N, D), jnp.float32),
    )(partials, W2, b2)


def kernel(X, edge_index, edge_weight, W, b):
    src = edge_index[0].astype(jnp.int32)
    dst = edge_index[1].astype(jnp.int32).reshape(NS, NCHUNK, CH)
    ew = edge_weight.reshape(NS, NCHUNK, CH)
    src2 = jnp.stack([src * 2, src * 2 + 1]).reshape(NC, NS, NCHUNK, CH)
    x2 = X.astype(jnp.bfloat16).reshape(NC * N, DH)
    partials = _aggregate(x2, src2, dst, ew)
    w2 = jnp.stack([W[_LPERM], W[DH + _LPERM]])
    return _finish(partials, w2, b.reshape(1, D))


# speedup vs baseline: 1.7790x; 1.0663x over previous
import functools

import jax
import jax.numpy as jnp
from jax import lax
from jax.experimental import pallas as pl
from jax.experimental.pallas import tpu as pltpu
from jax.experimental.pallas import tpu_sc as plsc

N = 10000
E = 320000
D = 128
NC = 2
NS = 16
NW = NC * NS
EPT = E // NW
CH = 80
NCHUNK = EPT // CH
NPROC = 126
NPADC = NPROC + 2
RPT = 624
RTAIL = N - NS * RPT

_mesh = plsc.VectorSubcoreMesh(core_axis_name="c", subcore_axis_name="s")


@functools.partial(
    pl.kernel,
    mesh=_mesh,
    compiler_params=pltpu.CompilerParams(
        use_tc_tiling_on_sc=False, needs_layout_passes=False),
    out_type=jax.ShapeDtypeStruct((NC, N, D), jnp.bfloat16),
    scratch_types=[
        pltpu.VMEM((NPADC, CH), jnp.int32),
        pltpu.VMEM((NPROC, CH), jnp.int32),
        pltpu.VMEM((CH, D), jnp.bfloat16),
        pltpu.VMEM((CH, D), jnp.bfloat16),
        pltpu.VMEM((CH, D), jnp.bfloat16),
        pltpu.VMEM((CH, D), jnp.bfloat16),
        pltpu.VMEM((NPROC, CH), jnp.float32),
        pltpu.VMEM_SHARED((N, D), jnp.bfloat16),
        pltpu.SemaphoreType.DMA,
        pltpu.SemaphoreType.DMA,
        pltpu.SemaphoreType.DMA,
        pltpu.SemaphoreType.DMA,
    ],
)
def _aggregate(xb_hbm, src_hbm, dst_hbm, ew_hbm, out_hbm,
               src_v, dst_v, g0_v, g1_v, s0_v, s1_v, ew_v,
               acc, sem0, sem1, ssem0, ssem1):
    cc = lax.axis_index("c")
    ss = lax.axis_index("s")
    t = cc * NS + ss

    pltpu.sync_copy(src_hbm.at[t], src_v)
    pltpu.sync_copy(dst_hbm.at[t], dst_v)
    pltpu.sync_copy(ew_hbm.at[t], ew_v)

    zb = jnp.zeros((32,), jnp.bfloat16)

    def _zrow(i, _):
        for j in range(D // 32):
            s0_v[i, pl.ds(j * 32, 32)] = zb
            s1_v[i, pl.ds(j * 32, 32)] = zb
        return 0
    lax.fori_loop(0, CH, _zrow, 0)
    for k in range(7):
        pltpu.sync_copy(s0_v, acc.at[pl.ds(ss * RPT + k * CH, CH)])
    pltpu.sync_copy(s0_v.at[pl.ds(0, RPT - 7 * CH)],
                    acc.at[pl.ds(ss * RPT + 7 * CH, RPT - 7 * CH)])

    @pl.when(ss == NS - 1)
    def _zero_tail():
        pltpu.sync_copy(s0_v.at[pl.ds(0, RTAIL)],
                        acc.at[pl.ds(NS * RPT, RTAIL)])

    plsc.subcore_barrier()

    himask = jnp.int32(-65536)

    def _scale(gbuf, sbuf, ci):
        def _grp(g, _):
            wvec = ew_v[ci, pl.ds(g * 16, 16)]
            for l in range(16):
                e = g * 16 + l
                w = wvec[l]
                for h in range(D // 32):
                    v = plsc.bitcast(gbuf[e, pl.ds(h * 32, 32)], jnp.int32)
                    lo = plsc.bitcast(v << 16, jnp.float32)
                    hi = plsc.bitcast(v & himask, jnp.float32)
                    sbuf[e, pl.ds(h * 32, 32)] = plsc.pack(
                        lo * w, hi * w, format=plsc.PackFormat.INTERLEAVED)
            return 0
        lax.fori_loop(0, CH // 16, _grp, 0)

    pltpu.async_copy(xb_hbm.at[src_v.at[0]], g0_v, sem0)
    pltpu.async_copy(xb_hbm.at[src_v.at[1]], g1_v, sem1)
    pltpu.async_copy(s0_v, acc.at[dst_v.at[0]], ssem0, add=True)
    pltpu.async_copy(s1_v, acc.at[dst_v.at[1]], ssem1, add=True)

    def _pair(i, _):
        c0 = i * 2
        pltpu.make_async_copy(xb_hbm.at[src_v.at[c0]], g0_v, sem0).wait()
        pltpu.make_async_copy(s0_v, acc.at[dst_v.at[c0]], ssem0).wait()
        _scale(g0_v, s0_v, c0)
        pltpu.async_copy(xb_hbm.at[src_v.at[c0 + 2]], g0_v, sem0)
        pltpu.async_copy(s0_v, acc.at[dst_v.at[c0]], ssem0, add=True)

        pltpu.make_async_copy(xb_hbm.at[src_v.at[c0 + 1]], g1_v, sem1).wait()
        pltpu.make_async_copy(s1_v, acc.at[dst_v.at[c0 + 1]], ssem1).wait()
        _scale(g1_v, s1_v, c0 + 1)
        pltpu.async_copy(xb_hbm.at[src_v.at[c0 + 3]], g1_v, sem1)
        pltpu.async_copy(s1_v, acc.at[dst_v.at[c0 + 1]], ssem1, add=True)
        return 0

    lax.fori_loop(0, NPROC // 2, _pair, 0)
    pltpu.make_async_copy(s0_v, acc.at[dst_v.at[0]], ssem0).wait()
    pltpu.make_async_copy(s1_v, acc.at[dst_v.at[1]], ssem1).wait()
    pltpu.make_async_copy(xb_hbm.at[src_v.at[NPROC]], g0_v, sem0).wait()
    pltpu.make_async_copy(xb_hbm.at[src_v.at[NPROC + 1]], g1_v, sem1).wait()
    plsc.subcore_barrier()

    pltpu.sync_copy(acc.at[pl.ds(ss * RPT, RPT)],
                    out_hbm.at[cc, pl.ds(ss * RPT, RPT)])

    @pl.when(ss == NS - 1)
    def _write_tail():
        pltpu.sync_copy(acc.at[pl.ds(NS * RPT, RTAIL)],
                        out_hbm.at[cc, pl.ds(NS * RPT, RTAIL)])


_BM = 1000


def _mm_body(p_ref, w_ref, b_ref, o_ref):
    o_ref[...] = (
        jnp.dot(p_ref[0], w_ref[...], preferred_element_type=jnp.float32)
        + jnp.dot(p_ref[1], w_ref[...], preferred_element_type=jnp.float32)
        + b_ref[...]
    )


def _finish(partials, W, b2):
    return pl.pallas_call(
        _mm_body,
        grid=(N // _BM,),
        in_specs=[
            pl.BlockSpec((NC, _BM, D), lambda i: (0, i, 0)),
            pl.BlockSpec((D, D), lambda i: (0, 0)),
            pl.BlockSpec((1, D), lambda i: (0, 0)),
        ],
        out_specs=pl.BlockSpec((_BM, D), lambda i: (i, 0)),
        out_shape=jax.ShapeDtypeStruct((N, D), jnp.float32),
    )(partials, W, b2)


def kernel(X, edge_index, edge_weight, W, b):
    src = jnp.pad(edge_index[0].astype(jnp.int32).reshape(NW, NCHUNK, CH),
                  ((0, 0), (0, NPADC - NCHUNK), (0, 0)))
    dst = jnp.pad(edge_index[1].astype(jnp.int32).reshape(NW, NCHUNK, CH),
                  ((0, 0), (0, NPROC - NCHUNK), (0, 0)))
    ew = jnp.pad(edge_weight.reshape(NW, NCHUNK, CH),
                 ((0, 0), (0, NPROC - NCHUNK), (0, 0)))
    partials = _aggregate(X.astype(jnp.bfloat16), src, dst, ew)
    return _finish(partials, W, b.reshape(1, D))
